# Initial kernel scaffold; baseline (speedup 1.0000x reference)
import jax
import jax.numpy as jnp
from jax.experimental import pallas as pl


def kernel(input, edge, edge_embed, edge_list_nhop, edge_embed_nhop, entity_rank, Corpus_, a, a_2):
    raise NotImplementedError("write your pallas kernel here")


# speedup vs baseline: 5.9839x; 5.9839x over previous
import functools

import jax
import jax.numpy as jnp
from jax import lax
from jax.experimental import pallas as pl
from jax.experimental.pallas import tpu as pltpu
from jax.experimental.pallas import tpu_sc as plsc

ALPHA = 0.2
DAMP = 0.85
NC = 2
NS = 16
NW = NC * NS
L = 16
CH = 80


def _lrelu(x):
    return jnp.maximum(x, ALPHA * x)


def _tc_pq(x, a_srcT, a_dstT):
    n, f = x.shape
    out = a_srcT.shape[1]
    bn = 2000

    def body(x_ref, as_ref, ad_ref, p_ref, q_ref):
        xb = x_ref[...]
        p_ref[...] = jnp.dot(xb, as_ref[...], preferred_element_type=jnp.float32)
        q_ref[...] = jnp.dot(xb, ad_ref[...], preferred_element_type=jnp.float32)

    return pl.pallas_call(
        body,
        grid=(n // bn,),
        in_specs=[
            pl.BlockSpec((bn, f), lambda i: (i, 0)),
            pl.BlockSpec((f, out), lambda i: (0, 0)),
            pl.BlockSpec((f, out), lambda i: (0, 0)),
        ],
        out_specs=[
            pl.BlockSpec((bn, out), lambda i: (i, 0)),
            pl.BlockSpec((bn, out), lambda i: (i, 0)),
        ],
        out_shape=[
            jax.ShapeDtypeStruct((n, out), jnp.float32),
            jax.ShapeDtypeStruct((n, out), jnp.float32),
        ],
    )(x, a_srcT, a_dstT)


def _tc_r(ee, a_relT):
    e, nr = ee.shape
    out = a_relT.shape[1]
    be = 8000

    def body(ee_ref, ar_ref, r_ref):
        r_ref[...] = jnp.dot(ee_ref[...], ar_ref[...],
                             preferred_element_type=jnp.float32)

    return pl.pallas_call(
        body,
        grid=(e // be,),
        in_specs=[
            pl.BlockSpec((be, nr), lambda i: (i, 0)),
            pl.BlockSpec((nr, out), lambda i: (0, 0)),
        ],
        out_specs=pl.BlockSpec((be, out), lambda i: (i, 0)),
        out_shape=jax.ShapeDtypeStruct((e, out), jnp.float32),
    )(ee, a_relT)


def _tc_post(hp):
    _, n, out = hp.shape
    bn = 2000

    def body(hp_ref, o_ref):
        h = hp_ref[0] + hp_ref[1]
        o_ref[...] = jnp.where(h > 0, h, jnp.exp(h) - 1.0)

    return pl.pallas_call(
        body,
        grid=(n // bn,),
        in_specs=[pl.BlockSpec((2, bn, out), lambda i: (0, i, 0))],
        out_specs=pl.BlockSpec((bn, out), lambda i: (i, 0)),
        out_shape=jax.ShapeDtypeStruct((n, out), jnp.float32),
    )(hp)


def _sc_edge_mlp(src, dst, p, q, r, a2):
    e_total = src.shape[0]
    out = p.shape[1]
    per_tile = e_total // NW
    n_chunks = per_tile // CH
    mesh = plsc.VectorSubcoreMesh(core_axis_name="c", subcore_axis_name="s")

    @functools.partial(
        pl.kernel,
        out_type=[
            jax.ShapeDtypeStruct((e_total, out), jnp.float32),
            jax.ShapeDtypeStruct((e_total,), jnp.float32),
        ],
        mesh=mesh,
        compiler_params=pltpu.CompilerParams(needs_layout_passes=False),
        scratch_types=[
            pltpu.VMEM((CH,), jnp.int32),
            pltpu.VMEM((CH,), jnp.int32),
            pltpu.VMEM((CH, 128), jnp.float32),
            pltpu.VMEM((CH, 128), jnp.float32),
            pltpu.VMEM((CH, 128), jnp.float32),
            pltpu.VMEM((CH, 128), jnp.float32),
            pltpu.VMEM((CH,), jnp.float32),
            pltpu.VMEM((128,), jnp.float32),
            pltpu.VMEM((L, L), jnp.float32),
            pltpu.SemaphoreType.DMA,
            pltpu.SemaphoreType.DMA,
            pltpu.SemaphoreType.DMA,
        ],
    )
    def kern(src_hbm, dst_hbm, p_hbm, q_hbm, r_hbm, a2_hbm, m_out, e_out,
             src_v, dst_v, p_v, q_v, r_v, m_v, e_v, a2_v, accbuf,
             sem1, sem2, sem3):
        wid = lax.axis_index("s") * NC + lax.axis_index("c")
        base = wid * per_tile
        pltpu.sync_copy(a2_hbm, a2_v)
        lane = lax.iota(jnp.int32, L)

        def chunk_body(i, carry):
            off = base + i * CH
            pltpu.sync_copy(src_hbm.at[pl.ds(off, CH)], src_v)
            pltpu.sync_copy(dst_hbm.at[pl.ds(off, CH)], dst_v)
            cp1 = pltpu.async_copy(p_hbm.at[src_v], p_v, sem1)
            cp2 = pltpu.async_copy(q_hbm.at[dst_v], q_v, sem2)
            cp3 = pltpu.async_copy(r_hbm.at[pl.ds(off, CH), :], r_v, sem3)
            cp1.wait()
            cp2.wait()
            cp3.wait()

            def edge16(k, carry2):
                def one(kk, carry3):
                    e_idx = k * L + kk
                    acc = jnp.zeros((L,), jnp.float32)
                    for j in range(8):
                        sl = pl.ds(j * L, L)
                        s = p_v[e_idx, sl] + q_v[e_idx, sl] + r_v[e_idx, sl]
                        m = _lrelu(s)
                        m_v[e_idx, sl] = m
                        acc = acc + a2_v[sl] * m
                    accbuf[kk, :] = acc
                    return carry3

                lax.fori_loop(0, L, one, 0)
                tot = jnp.zeros((L,), jnp.float32)
                for cc in range(L):
                    col = plsc.load_gather(
                        accbuf, [lane, jnp.full((L,), cc, jnp.int32)])
                    tot = tot + col
                z = jnp.minimum(-tot, -ALPHA * tot)
                e_v[pl.ds(k * L, L)] = jnp.exp(z)
                return carry2

            lax.fori_loop(0, CH // L, edge16, 0)
            pltpu.sync_copy(m_v, m_out.at[pl.ds(off, CH), :])
            pltpu.sync_copy(e_v, e_out.at[pl.ds(off, CH)])
            return carry

        lax.fori_loop(0, n_chunks, chunk_body, 0)

    return kern(src, dst, p, q, r, a2)


def _sc_aggregate(src4d, dst4d, e4d, rank_pad, m, zeros_n, zeros_h):
    _, ng, gsz, chw = src4d.shape
    n_nodes = 10000
    out = m.shape[1]
    n_pad = rank_pad.shape[0]
    npt = n_pad // NS
    rows_per_tile = ng * gsz
    mesh = plsc.VectorSubcoreMesh(core_axis_name="c", subcore_axis_name="s")

    @functools.partial(
        pl.kernel,
        out_type=[
            jax.ShapeDtypeStruct((2, n_nodes, out), jnp.float32),
            jax.ShapeDtypeStruct((n_nodes,), jnp.float32),
            jax.ShapeDtypeStruct((NS, ng, gsz, chw), jnp.float32),
        ],
        mesh=mesh,
        compiler_params=pltpu.CompilerParams(needs_layout_passes=False),
        scratch_types=[
            pltpu.VMEM((gsz, CH), jnp.int32),
            pltpu.VMEM((gsz, CH), jnp.int32),
            pltpu.VMEM((gsz, CH), jnp.float32),
            pltpu.VMEM((gsz, CH), jnp.float32),
            pltpu.VMEM((CH,), jnp.float32),
            pltpu.VMEM((n_pad,), jnp.float32),
            pltpu.VMEM((npt,), jnp.float32),
            pltpu.VMEM((npt,), jnp.float32),
            pltpu.VMEM((CH, 128), jnp.float32),
            pltpu.VMEM_SHARED((n_pad,), jnp.float32),
            pltpu.VMEM_SHARED((n_pad,), jnp.float32),
            pltpu.VMEM_SHARED((n_pad,), jnp.float32),
            pltpu.VMEM_SHARED((n_pad, 128), jnp.float32),
        ],
    )
    def kern(src_hbm, dst_hbm, e_hbm, rank_hbm, m_hbm, zn_hbm, zh_hbm,
             hp_out, rn_out, rel_out,
             src_b, dst_b, e_b, rel_b, wbuf, tbl, cb1, cb2, mrow_v,
             x_sp, y_sp, z_sp, h_sp):
        c = lax.axis_index("c")
        s = lax.axis_index("s")
        row0 = s * rows_per_tile
        nd0 = s * npt
        ndsl = pl.ds(nd0, npt)

        pltpu.sync_copy(zn_hbm, x_sp.at[ndsl])
        pltpu.sync_copy(zn_hbm, y_sp.at[ndsl])
        pltpu.sync_copy(zn_hbm, z_sp.at[ndsl])
        pltpu.sync_copy(zh_hbm, h_sp.at[ndsl, :])
        plsc.subcore_barrier()

        def p1(g, carry):
            pltpu.sync_copy(e_hbm.at[s, g], e_b)
            pltpu.sync_copy(dst_hbm.at[s, g], dst_b)
            def p1j(j, carry2):
                pltpu.sync_copy(e_b.at[j], x_sp.at[dst_b.at[j]], add=True)
                return carry2
            lax.fori_loop(0, gsz, p1j, 0)
            return carry
        lax.fori_loop(0, ng, p1, 0)
        plsc.subcore_barrier()

        pltpu.sync_copy(x_sp.at[ndsl], cb1)
        def p1b(k, carry):
            sl = pl.ds(k * L, L)
            v = cb1[sl]
            v = jnp.where(v == 0.0, jnp.float32(1e-12), v)
            cb1[sl] = jnp.float32(1.0) / v
            return carry
        lax.fori_loop(0, npt // L, p1b, 0)
        pltpu.sync_copy(cb1, x_sp.at[ndsl])
        plsc.subcore_barrier()

        pltpu.sync_copy(x_sp, tbl)
        def p2(g, carry):
            pltpu.sync_copy(e_hbm.at[s, g], e_b)
            pltpu.sync_copy(dst_hbm.at[s, g], dst_b)
            pltpu.sync_copy(src_hbm.at[s, g], src_b)
            def p2j(j, carry2):
                for k in range(CH // L):
                    sl = pl.ds(k * L, L)
                    av = plsc.load_gather(tbl, [dst_b[j, sl]])
                    rel_b[j, sl] = e_b[j, sl] * av
                pltpu.sync_copy(rel_b.at[j], y_sp.at[src_b.at[j]], add=True)
                return carry2
            lax.fori_loop(0, gsz, p2j, 0)
            pltpu.sync_copy(rel_b, rel_out.at[s, g])
            return carry
        lax.fori_loop(0, ng, p2, 0)
        plsc.subcore_barrier()

        pltpu.sync_copy(y_sp.at[ndsl], cb1)
        pltpu.sync_copy(rank_hbm.at[ndsl], cb2)
        def p2b(k, carry):
            sl = pl.ds(k * L, L)
            v = cb1[sl]
            v = jnp.where(v == 0.0, jnp.float32(1e-12), v)
            cb1[sl] = cb2[sl] / v
            return carry
        lax.fori_loop(0, npt // L, p2b, 0)
        pltpu.sync_copy(cb1, y_sp.at[ndsl])
        plsc.subcore_barrier()

        pltpu.sync_copy(y_sp, tbl)
        def p3(g, carry):
            pltpu.sync_copy(rel_out.at[s, g], rel_b)
            pltpu.sync_copy(dst_hbm.at[s, g], dst_b)
            pltpu.sync_copy(src_hbm.at[s, g], src_b)
            def p3j(j, carry2):
                for k in range(CH // L):
                    sl = pl.ds(k * L, L)
                    yv = plsc.load_gather(tbl, [src_b[j, sl]])
                    e_b[j, sl] = rel_b[j, sl] * yv
                pltpu.sync_copy(e_b.at[j], z_sp.at[dst_b.at[j]], add=True)
                return carry2
            lax.fori_loop(0, gsz, p3j, 0)
            return carry
        lax.fori_loop(0, ng, p3, 0)
        plsc.subcore_barrier()

        pltpu.sync_copy(z_sp.at[ndsl], cb1)
        def p3b(k, carry):
            sl = pl.ds(k * L, L)
            cb1[sl] = jnp.float32(1.0 - DAMP) + jnp.float32(DAMP) * cb1[sl]
            return carry
        lax.fori_loop(0, npt // L, p3b, 0)
        pltpu.sync_copy(cb1, z_sp.at[ndsl])

        @pl.when(c == 0)
        def _():
            @pl.when(s < NS - 1)
            def _():
                pltpu.sync_copy(cb1, rn_out.at[ndsl])
            @pl.when(s == NS - 1)
            def _():
                rem = n_nodes - (NS - 1) * npt
                pltpu.sync_copy(cb1.at[pl.ds(0, rem)],
                                rn_out.at[pl.ds((NS - 1) * npt, rem)])
        plsc.subcore_barrier()

        pltpu.sync_copy(z_sp, tbl)
        def p6(g, carry):
            pltpu.sync_copy(rel_out.at[s, g], rel_b)
            pltpu.sync_copy(dst_hbm.at[s, g], dst_b)
            pltpu.sync_copy(src_hbm.at[s, g], src_b)
            def p6j(j, carry2):
                @pl.when(j % 2 == c)
                def _():
                    i = g * gsz + j
                    pltpu.sync_copy(m_hbm.at[pl.ds((row0 + i) * CH, CH), :],
                                    mrow_v)
                    for k in range(CH // L):
                        sl = pl.ds(k * L, L)
                        zv = plsc.load_gather(tbl, [src_b[j, sl]])
                        wbuf[sl] = rel_b[j, sl] * zv

                    def rowloop(row, carry3):
                        wb = plsc.load_gather(
                            wbuf, [jnp.full((L,), row, jnp.int32)])
                        for jj in range(8):
                            fsl = pl.ds(jj * L, L)
                            mrow_v[row, fsl] = mrow_v[row, fsl] * wb
                        return carry3
                    lax.fori_loop(0, CH, rowloop, 0)
                    pltpu.sync_copy(mrow_v, h_sp.at[dst_b.at[j]], add=True)
                return carry2
            lax.fori_loop(0, gsz, p6j, 0)
            return carry
        lax.fori_loop(0, ng, p6, 0)
        plsc.subcore_barrier()

        @pl.when(s < NS - 1)
        def _():
            pltpu.sync_copy(h_sp.at[ndsl, :], hp_out.at[c, ndsl, :])
        @pl.when(s == NS - 1)
        def _():
            rem = n_nodes - (NS - 1) * npt
            pltpu.sync_copy(h_sp.at[pl.ds((NS - 1) * npt, rem), :],
                            hp_out.at[c, pl.ds((NS - 1) * npt, rem), :])

    return kern(src4d, dst4d, e4d, rank_pad, m, zeros_n, zeros_h)


def kernel(input, edge, edge_embed, edge_list_nhop, edge_embed_nhop,
           entity_rank, Corpus_, a, a_2):
    x = input
    n, in_f = x.shape
    edge_all = jnp.concatenate([edge, edge_list_nhop], axis=1)
    ee_all = jnp.concatenate([edge_embed, edge_embed_nhop], axis=0)
    e_total = edge_all.shape[1]
    nrela = ee_all.shape[1]
    out_f = a.shape[0]

    src = edge_all[0]
    dst = edge_all[1]

    a_srcT = a[:, :in_f].T
    a_dstT = a[:, in_f:2 * in_f].T
    a_relT = a[:, 2 * in_f:].T
    a2_vec = a_2.reshape(out_f)

    p_arr, q_arr = _tc_pq(x, a_srcT, a_dstT)
    r_arr = _tc_r(ee_all, a_relT)

    m_arr, e_arr = _sc_edge_mlp(src, dst, p_arr, q_arr, r_arr, a2_vec)

    gsz = 10
    ngrp = e_total // CH // NS // gsz
    shape4 = (NS, ngrp, gsz, CH)
    src4d = src.reshape(shape4)
    dst4d = dst.reshape(shape4)
    e4d = e_arr.reshape(shape4)
    n_pad = 10240
    rank_pad = jnp.concatenate(
        [entity_rank, jnp.zeros((n_pad - n,), jnp.float32)])
    zeros_n = jnp.zeros((n_pad // NS,), jnp.float32)
    zeros_h = jnp.zeros((n_pad // NS, out_f), jnp.float32)

    hp, rank_new, _ = _sc_aggregate(src4d, dst4d, e4d, rank_pad, m_arr,
                                    zeros_n, zeros_h)
    h_out = _tc_post(hp)
    return (h_out, rank_new)


# speedup vs baseline: 7.3790x; 1.2331x over previous
import functools

import jax
import jax.numpy as jnp
from jax import lax
from jax.experimental import pallas as pl
from jax.experimental.pallas import tpu as pltpu
from jax.experimental.pallas import tpu_sc as plsc

ALPHA = 0.2
DAMP = 0.85
NC = 2
NS = 16
NW = NC * NS
L = 16
CH = 80


def _lrelu(x):
    return jnp.maximum(x, ALPHA * x)


def _tc_pq(x, a_srcT, a_dstT):
    n, f = x.shape
    out = a_srcT.shape[1]
    bn = 2000

    def body(x_ref, as_ref, ad_ref, p_ref, q_ref):
        xb = x_ref[...]
        p_ref[...] = jnp.dot(xb, as_ref[...], preferred_element_type=jnp.float32)
        q_ref[...] = jnp.dot(xb, ad_ref[...], preferred_element_type=jnp.float32)

    return pl.pallas_call(
        body,
        grid=(n // bn,),
        in_specs=[
            pl.BlockSpec((bn, f), lambda i: (i, 0)),
            pl.BlockSpec((f, out), lambda i: (0, 0)),
            pl.BlockSpec((f, out), lambda i: (0, 0)),
        ],
        out_specs=[
            pl.BlockSpec((bn, out), lambda i: (i, 0)),
            pl.BlockSpec((bn, out), lambda i: (i, 0)),
        ],
        out_shape=[
            jax.ShapeDtypeStruct((n, out), jnp.float32),
            jax.ShapeDtypeStruct((n, out), jnp.float32),
        ],
    )(x, a_srcT, a_dstT)


def _tc_r(ee, a_relT):
    e, nr = ee.shape
    out = a_relT.shape[1]
    be = 8000

    def body(ee_ref, ar_ref, r_ref):
        r_ref[...] = jnp.dot(ee_ref[...], ar_ref[...],
                             preferred_element_type=jnp.float32)

    return pl.pallas_call(
        body,
        grid=(e // be,),
        in_specs=[
            pl.BlockSpec((be, nr), lambda i: (i, 0)),
            pl.BlockSpec((nr, out), lambda i: (0, 0)),
        ],
        out_specs=pl.BlockSpec((be, out), lambda i: (i, 0)),
        out_shape=jax.ShapeDtypeStruct((e, out), jnp.float32),
    )(ee, a_relT)


def _tc_post(hp):
    _, n, out = hp.shape
    bn = 2000

    def body(hp_ref, o_ref):
        h = hp_ref[0] + hp_ref[1]
        o_ref[...] = jnp.where(h > 0, h, jnp.exp(h) - 1.0)

    return pl.pallas_call(
        body,
        grid=(n // bn,),
        in_specs=[pl.BlockSpec((2, bn, out), lambda i: (0, i, 0))],
        out_specs=pl.BlockSpec((bn, out), lambda i: (i, 0)),
        out_shape=jax.ShapeDtypeStruct((n, out), jnp.float32),
    )(hp)


def _sc_edge_mlp(src3, dst3, p, q, r, a2):
    _, n_chunks, chw = src3.shape
    e_total = NW * n_chunks * chw
    out = p.shape[1]
    mesh = plsc.VectorSubcoreMesh(core_axis_name="c", subcore_axis_name="s")

    @functools.partial(
        pl.kernel,
        out_type=[
            jax.ShapeDtypeStruct((e_total, out), jnp.float32),
            jax.ShapeDtypeStruct((e_total,), jnp.float32),
        ],
        mesh=mesh,
        compiler_params=pltpu.CompilerParams(needs_layout_passes=False),
        scratch_types=[
            pltpu.VMEM((n_chunks, CH), jnp.int32),
            pltpu.VMEM((n_chunks, CH), jnp.int32),
            [pltpu.VMEM((CH, 128), jnp.float32)] * 2,
            [pltpu.VMEM((CH, 128), jnp.float32)] * 2,
            [pltpu.VMEM((CH, 128), jnp.float32)] * 2,
            [pltpu.VMEM((CH, 128), jnp.float32)] * 2,
            [pltpu.VMEM((CH,), jnp.float32)] * 2,
            pltpu.VMEM((128,), jnp.float32),
            pltpu.VMEM((L, L), jnp.float32),
            [pltpu.SemaphoreType.DMA] * 2,
            [pltpu.SemaphoreType.DMA] * 2,
        ],
    )
    def kern(src_hbm, dst_hbm, p_hbm, q_hbm, r_hbm, a2_hbm, m_out, e_out,
             srcb, dstb, p_v, q_v, r_v, m_v, e_v, a2_v, accbuf,
             semg, semw):
        wid = lax.axis_index("s") * NC + lax.axis_index("c")
        base = wid * n_chunks * CH
        pltpu.sync_copy(a2_hbm, a2_v)
        pltpu.sync_copy(src_hbm.at[wid], srcb)
        pltpu.sync_copy(dst_hbm.at[wid], dstb)
        lane = lax.iota(jnp.int32, L)

        def issue_gather(i, b):
            pltpu.async_copy(p_hbm.at[srcb.at[i]], p_v[b], semg[b])
            pltpu.async_copy(q_hbm.at[dstb.at[i]], q_v[b], semg[b])
            pltpu.async_copy(r_hbm.at[pl.ds(base + i * CH, CH), :],
                             r_v[b], semg[b])

        def wait_gather(b):
            pltpu.make_async_copy(p_hbm.at[srcb.at[0]], p_v[b], semg[b]).wait()
            pltpu.make_async_copy(q_hbm.at[dstb.at[0]], q_v[b], semg[b]).wait()
            pltpu.make_async_copy(r_hbm.at[pl.ds(base, CH), :],
                                  r_v[b], semg[b]).wait()

        def wait_write(b):
            pltpu.make_async_copy(m_v[b], m_out.at[pl.ds(base, CH), :],
                                  semw[b]).wait()
            pltpu.make_async_copy(e_v[b], e_out.at[pl.ds(base, CH)],
                                  semw[b]).wait()

        def compute_store(i, b):
            pv, qv, rv, mv, ev = p_v[b], q_v[b], r_v[b], m_v[b], e_v[b]

            def edge16(k, carry2):
                def one(kk, carry3):
                    e_idx = k * L + kk
                    acc = jnp.zeros((L,), jnp.float32)
                    for j in range(8):
                        sl = pl.ds(j * L, L)
                        s = pv[e_idx, sl] + qv[e_idx, sl] + rv[e_idx, sl]
                        m = _lrelu(s)
                        mv[e_idx, sl] = m
                        acc = acc + a2_v[sl] * m
                    accbuf[kk, :] = acc
                    return carry3

                lax.fori_loop(0, L, one, 0)
                tot = jnp.zeros((L,), jnp.float32)
                for cc in range(L):
                    col = plsc.load_gather(
                        accbuf, [lane, jnp.full((L,), cc, jnp.int32)])
                    tot = tot + col
                z = jnp.minimum(-tot, -ALPHA * tot)
                ev[pl.ds(k * L, L)] = jnp.exp(z)
                return carry2

            lax.fori_loop(0, CH // L, edge16, 0)
            off = base + i * CH
            pltpu.async_copy(mv, m_out.at[pl.ds(off, CH), :], semw[b])
            pltpu.async_copy(ev, e_out.at[pl.ds(off, CH)], semw[b])

        issue_gather(0, 0)

        def pair(g, carry):
            i0 = 2 * g
            i1 = i0 + 1

            @pl.when(i1 < n_chunks)
            def _():
                issue_gather(i1, 1)
            wait_gather(0)

            @pl.when(g > 0)
            def _():
                wait_write(0)
            compute_store(i0, 0)

            @pl.when(i0 + 2 < n_chunks)
            def _():
                issue_gather(i0 + 2, 0)

            @pl.when(i1 < n_chunks)
            def _():
                wait_gather(1)

                @pl.when(g > 0)
                def _():
                    wait_write(1)
                compute_store(i1, 1)
            return carry

        lax.fori_loop(0, (n_chunks + 1) // 2, pair, 0)
        wait_write(0)
        wait_write(1)

    return kern(src3, dst3, p, q, r, a2)


def _sc_aggregate(src4d, dst4d, e4d, rank_pad, m, zeros_n, zeros_h):
    _, ng, gsz, chw = src4d.shape
    n_nodes = 10000
    out = m.shape[1]
    n_pad = rank_pad.shape[0]
    npt = n_pad // NS
    rows_per_tile = ng * gsz
    mesh = plsc.VectorSubcoreMesh(core_axis_name="c", subcore_axis_name="s")

    @functools.partial(
        pl.kernel,
        out_type=[
            jax.ShapeDtypeStruct((2, n_nodes, out), jnp.float32),
            jax.ShapeDtypeStruct((n_nodes,), jnp.float32),
            jax.ShapeDtypeStruct((NS, ng, gsz, chw), jnp.float32),
        ],
        mesh=mesh,
        compiler_params=pltpu.CompilerParams(needs_layout_passes=False),
        scratch_types=[
            pltpu.VMEM((gsz, CH), jnp.int32),
            pltpu.VMEM((gsz, CH), jnp.int32),
            pltpu.VMEM((gsz, CH), jnp.float32),
            pltpu.VMEM((gsz, CH), jnp.float32),
            pltpu.VMEM((CH,), jnp.float32),
            pltpu.VMEM((n_pad,), jnp.float32),
            pltpu.VMEM((npt,), jnp.float32),
            pltpu.VMEM((npt,), jnp.float32),
            pltpu.VMEM((CH, 128), jnp.float32),
            pltpu.VMEM_SHARED((n_pad,), jnp.float32),
            pltpu.VMEM_SHARED((n_pad,), jnp.float32),
            pltpu.VMEM_SHARED((n_pad,), jnp.float32),
            pltpu.VMEM_SHARED((n_pad, 128), jnp.float32),
        ],
    )
    def kern(src_hbm, dst_hbm, e_hbm, rank_hbm, m_hbm, zn_hbm, zh_hbm,
             hp_out, rn_out, rel_out,
             src_b, dst_b, e_b, rel_b, wbuf, tbl, cb1, cb2, mrow_v,
             x_sp, y_sp, z_sp, h_sp):
        c = lax.axis_index("c")
        s = lax.axis_index("s")
        row0 = s * rows_per_tile
        nd0 = s * npt
        ndsl = pl.ds(nd0, npt)

        pltpu.sync_copy(zn_hbm, x_sp.at[ndsl])
        pltpu.sync_copy(zn_hbm, y_sp.at[ndsl])
        pltpu.sync_copy(zn_hbm, z_sp.at[ndsl])
        pltpu.sync_copy(zh_hbm, h_sp.at[ndsl, :])
        plsc.subcore_barrier()

        def p1(g, carry):
            pltpu.sync_copy(e_hbm.at[s, g], e_b)
            pltpu.sync_copy(dst_hbm.at[s, g], dst_b)
            def p1j(j, carry2):
                pltpu.sync_copy(e_b.at[j], x_sp.at[dst_b.at[j]], add=True)
                return carry2
            lax.fori_loop(0, gsz, p1j, 0)
            return carry
        lax.fori_loop(0, ng, p1, 0)
        plsc.subcore_barrier()

        pltpu.sync_copy(x_sp.at[ndsl], cb1)
        def p1b(k, carry):
            sl = pl.ds(k * L, L)
            v = cb1[sl]
            v = jnp.where(v == 0.0, jnp.float32(1e-12), v)
            cb1[sl] = jnp.float32(1.0) / v
            return carry
        lax.fori_loop(0, npt // L, p1b, 0)
        pltpu.sync_copy(cb1, x_sp.at[ndsl])
        plsc.subcore_barrier()

        pltpu.sync_copy(x_sp, tbl)
        def p2(g, carry):
            pltpu.sync_copy(e_hbm.at[s, g], e_b)
            pltpu.sync_copy(dst_hbm.at[s, g], dst_b)
            pltpu.sync_copy(src_hbm.at[s, g], src_b)
            def p2j(j, carry2):
                for k in range(CH // L):
                    sl = pl.ds(k * L, L)
                    av = plsc.load_gather(tbl, [dst_b[j, sl]])
                    rel_b[j, sl] = e_b[j, sl] * av
                pltpu.sync_copy(rel_b.at[j], y_sp.at[src_b.at[j]], add=True)
                return carry2
            lax.fori_loop(0, gsz, p2j, 0)
            pltpu.sync_copy(rel_b, rel_out.at[s, g])
            return carry
        lax.fori_loop(0, ng, p2, 0)
        plsc.subcore_barrier()

        pltpu.sync_copy(y_sp.at[ndsl], cb1)
        pltpu.sync_copy(rank_hbm.at[ndsl], cb2)
        def p2b(k, carry):
            sl = pl.ds(k * L, L)
            v = cb1[sl]
            v = jnp.where(v == 0.0, jnp.float32(1e-12), v)
            cb1[sl] = cb2[sl] / v
            return carry
        lax.fori_loop(0, npt // L, p2b, 0)
        pltpu.sync_copy(cb1, y_sp.at[ndsl])
        plsc.subcore_barrier()

        pltpu.sync_copy(y_sp, tbl)
        def p3(g, carry):
            pltpu.sync_copy(rel_out.at[s, g], rel_b)
            pltpu.sync_copy(dst_hbm.at[s, g], dst_b)
            pltpu.sync_copy(src_hbm.at[s, g], src_b)
            def p3j(j, carry2):
                for k in range(CH // L):
                    sl = pl.ds(k * L, L)
                    yv = plsc.load_gather(tbl, [src_b[j, sl]])
                    e_b[j, sl] = rel_b[j, sl] * yv
                pltpu.sync_copy(e_b.at[j], z_sp.at[dst_b.at[j]], add=True)
                return carry2
            lax.fori_loop(0, gsz, p3j, 0)
            return carry
        lax.fori_loop(0, ng, p3, 0)
        plsc.subcore_barrier()

        pltpu.sync_copy(z_sp.at[ndsl], cb1)
        def p3b(k, carry):
            sl = pl.ds(k * L, L)
            cb1[sl] = jnp.float32(1.0 - DAMP) + jnp.float32(DAMP) * cb1[sl]
            return carry
        lax.fori_loop(0, npt // L, p3b, 0)
        pltpu.sync_copy(cb1, z_sp.at[ndsl])

        @pl.when(c == 0)
        def _():
            @pl.when(s < NS - 1)
            def _():
                pltpu.sync_copy(cb1, rn_out.at[ndsl])
            @pl.when(s == NS - 1)
            def _():
                rem = n_nodes - (NS - 1) * npt
                pltpu.sync_copy(cb1.at[pl.ds(0, rem)],
                                rn_out.at[pl.ds((NS - 1) * npt, rem)])
        plsc.subcore_barrier()

        pltpu.sync_copy(z_sp, tbl)
        def p6(g, carry):
            pltpu.sync_copy(rel_out.at[s, g], rel_b)
            pltpu.sync_copy(dst_hbm.at[s, g], dst_b)
            pltpu.sync_copy(src_hbm.at[s, g], src_b)
            def p6j(j, carry2):
                @pl.when(j % 2 == c)
                def _():
                    i = g * gsz + j
                    pltpu.sync_copy(m_hbm.at[pl.ds((row0 + i) * CH, CH), :],
                                    mrow_v)
                    for k in range(CH // L):
                        sl = pl.ds(k * L, L)
                        zv = plsc.load_gather(tbl, [src_b[j, sl]])
                        wbuf[sl] = rel_b[j, sl] * zv

                    def rowloop(row, carry3):
                        wb = plsc.load_gather(
                            wbuf, [jnp.full((L,), row, jnp.int32)])
                        for jj in range(8):
                            fsl = pl.ds(jj * L, L)
                            mrow_v[row, fsl] = mrow_v[row, fsl] * wb
                        return carry3
                    lax.fori_loop(0, CH, rowloop, 0)
                    pltpu.sync_copy(mrow_v, h_sp.at[dst_b.at[j]], add=True)
                return carry2
            lax.fori_loop(0, gsz, p6j, 0)
            return carry
        lax.fori_loop(0, ng, p6, 0)
        plsc.subcore_barrier()

        @pl.when(s < NS - 1)
        def _():
            pltpu.sync_copy(h_sp.at[ndsl, :], hp_out.at[c, ndsl, :])
        @pl.when(s == NS - 1)
        def _():
            rem = n_nodes - (NS - 1) * npt
            pltpu.sync_copy(h_sp.at[pl.ds((NS - 1) * npt, rem), :],
                            hp_out.at[c, pl.ds((NS - 1) * npt, rem), :])

    return kern(src4d, dst4d, e4d, rank_pad, m, zeros_n, zeros_h)


def kernel(input, edge, edge_embed, edge_list_nhop, edge_embed_nhop,
           entity_rank, Corpus_, a, a_2):
    x = input
    n, in_f = x.shape
    edge_all = jnp.concatenate([edge, edge_list_nhop], axis=1)
    ee_all = jnp.concatenate([edge_embed, edge_embed_nhop], axis=0)
    e_total = edge_all.shape[1]
    nrela = ee_all.shape[1]
    out_f = a.shape[0]

    src = edge_all[0]
    dst = edge_all[1]

    a_srcT = a[:, :in_f].T
    a_dstT = a[:, in_f:2 * in_f].T
    a_relT = a[:, 2 * in_f:].T
    a2_vec = a_2.reshape(out_f)

    p_arr, q_arr = _tc_pq(x, a_srcT, a_dstT)
    r_arr = _tc_r(ee_all, a_relT)

    nch_a = e_total // NW // CH
    m_arr, e_arr = _sc_edge_mlp(src.reshape(NW, nch_a, CH),
                                dst.reshape(NW, nch_a, CH),
                                p_arr, q_arr, r_arr, a2_vec)

    gsz = 10
    ngrp = e_total // CH // NS // gsz
    shape4 = (NS, ngrp, gsz, CH)
    src4d = src.reshape(shape4)
    dst4d = dst.reshape(shape4)
    e4d = e_arr.reshape(shape4)
    n_pad = 10240
    rank_pad = jnp.concatenate(
        [entity_rank, jnp.zeros((n_pad - n,), jnp.float32)])
    zeros_n = jnp.zeros((n_pad // NS,), jnp.float32)
    zeros_h = jnp.zeros((n_pad // NS, out_f), jnp.float32)

    hp, rank_new, _ = _sc_aggregate(src4d, dst4d, e4d, rank_pad, m_arr,
                                    zeros_n, zeros_h)
    h_out = _tc_post(hp)
    return (h_out, rank_new)


# speedup vs baseline: 7.7853x; 1.0551x over previous
import functools

import jax
import jax.numpy as jnp
from jax import lax
from jax.experimental import pallas as pl
from jax.experimental.pallas import tpu as pltpu
from jax.experimental.pallas import tpu_sc as plsc

ALPHA = 0.2
DAMP = 0.85
NC = 2
NS = 16
NW = NC * NS
L = 16
CH = 80


def _lrelu(x):
    return jnp.maximum(x, ALPHA * x)


def _tc_pq(x, a_srcT, a_dstT):
    n, f = x.shape
    out = a_srcT.shape[1]
    bn = 2000

    def body(x_ref, as_ref, ad_ref, p_ref, q_ref):
        xb = x_ref[...]
        p_ref[...] = jnp.dot(xb, as_ref[...], preferred_element_type=jnp.float32)
        q_ref[...] = jnp.dot(xb, ad_ref[...], preferred_element_type=jnp.float32)

    return pl.pallas_call(
        body,
        grid=(n // bn,),
        in_specs=[
            pl.BlockSpec((bn, f), lambda i: (i, 0)),
            pl.BlockSpec((f, out), lambda i: (0, 0)),
            pl.BlockSpec((f, out), lambda i: (0, 0)),
        ],
        out_specs=[
            pl.BlockSpec((bn, out), lambda i: (i, 0)),
            pl.BlockSpec((bn, out), lambda i: (i, 0)),
        ],
        out_shape=[
            jax.ShapeDtypeStruct((n, out), jnp.float32),
            jax.ShapeDtypeStruct((n, out), jnp.float32),
        ],
    )(x, a_srcT, a_dstT)


def _tc_r(ee, a_relT):
    e, nr = ee.shape
    out = a_relT.shape[1]
    be = 8000

    def body(ee_ref, ar_ref, r_ref):
        r_ref[...] = jnp.dot(ee_ref[...], ar_ref[...],
                             preferred_element_type=jnp.float32)

    return pl.pallas_call(
        body,
        grid=(e // be,),
        in_specs=[
            pl.BlockSpec((be, nr), lambda i: (i, 0)),
            pl.BlockSpec((nr, out), lambda i: (0, 0)),
        ],
        out_specs=pl.BlockSpec((be, out), lambda i: (i, 0)),
        out_shape=jax.ShapeDtypeStruct((e, out), jnp.float32),
    )(ee, a_relT)


def _tc_post(hp):
    _, n, out = hp.shape
    bn = 2000

    def body(hp_ref, o_ref):
        h = hp_ref[0] + hp_ref[1]
        o_ref[...] = jnp.where(h > 0, h, jnp.exp(h) - 1.0)

    return pl.pallas_call(
        body,
        grid=(n // bn,),
        in_specs=[pl.BlockSpec((2, bn, out), lambda i: (0, i, 0))],
        out_specs=pl.BlockSpec((bn, out), lambda i: (i, 0)),
        out_shape=jax.ShapeDtypeStruct((n, out), jnp.float32),
    )(hp)


def _sc_edge_mlp(src3, dst3, p, q, r, a2):
    _, n_chunks, chw = src3.shape
    e_total = NW * n_chunks * chw
    out = p.shape[1]
    mesh = plsc.VectorSubcoreMesh(core_axis_name="c", subcore_axis_name="s")

    @functools.partial(
        pl.kernel,
        out_type=[
            jax.ShapeDtypeStruct((e_total, out), jnp.float32),
            jax.ShapeDtypeStruct((e_total,), jnp.float32),
        ],
        mesh=mesh,
        compiler_params=pltpu.CompilerParams(needs_layout_passes=False),
        scratch_types=[
            pltpu.VMEM((n_chunks, CH), jnp.int32),
            pltpu.VMEM((n_chunks, CH), jnp.int32),
            [pltpu.VMEM((CH, 128), jnp.float32)] * 2,
            [pltpu.VMEM((CH, 128), jnp.float32)] * 2,
            [pltpu.VMEM((CH, 128), jnp.float32)] * 2,
            [pltpu.VMEM((CH, 128), jnp.float32)] * 2,
            [pltpu.VMEM((CH,), jnp.float32)] * 2,
            pltpu.VMEM((128,), jnp.float32),
            pltpu.VMEM((L, L), jnp.float32),
            [pltpu.SemaphoreType.DMA] * 2,
            [pltpu.SemaphoreType.DMA] * 2,
        ],
    )
    def kern(src_hbm, dst_hbm, p_hbm, q_hbm, r_hbm, a2_hbm, m_out, e_out,
             srcb, dstb, p_v, q_v, r_v, m_v, e_v, a2_v, accbuf,
             semg, semw):
        wid = lax.axis_index("s") * NC + lax.axis_index("c")
        base = wid * n_chunks * CH
        pltpu.sync_copy(a2_hbm, a2_v)
        pltpu.sync_copy(src_hbm.at[wid], srcb)
        pltpu.sync_copy(dst_hbm.at[wid], dstb)
        lane = lax.iota(jnp.int32, L)

        def issue_gather(i, b):
            pltpu.async_copy(p_hbm.at[srcb.at[i]], p_v[b], semg[b])
            pltpu.async_copy(q_hbm.at[dstb.at[i]], q_v[b], semg[b])
            pltpu.async_copy(r_hbm.at[pl.ds(base + i * CH, CH), :],
                             r_v[b], semg[b])

        def wait_gather(b):
            pltpu.make_async_copy(p_hbm.at[srcb.at[0]], p_v[b], semg[b]).wait()
            pltpu.make_async_copy(q_hbm.at[dstb.at[0]], q_v[b], semg[b]).wait()
            pltpu.make_async_copy(r_hbm.at[pl.ds(base, CH), :],
                                  r_v[b], semg[b]).wait()

        def wait_write(b):
            pltpu.make_async_copy(m_v[b], m_out.at[pl.ds(base, CH), :],
                                  semw[b]).wait()
            pltpu.make_async_copy(e_v[b], e_out.at[pl.ds(base, CH)],
                                  semw[b]).wait()

        def compute_store(i, b):
            pv, qv, rv, mv, ev = p_v[b], q_v[b], r_v[b], m_v[b], e_v[b]

            def edge16(k, carry2):
                def one(kk, carry3):
                    e_idx = k * L + kk
                    acc = jnp.zeros((L,), jnp.float32)
                    for j in range(8):
                        sl = pl.ds(j * L, L)
                        s = pv[e_idx, sl] + qv[e_idx, sl] + rv[e_idx, sl]
                        m = _lrelu(s)
                        mv[e_idx, sl] = m
                        acc = acc + a2_v[sl] * m
                    accbuf[kk, :] = acc
                    return carry3

                lax.fori_loop(0, L, one, 0)
                tot = jnp.zeros((L,), jnp.float32)
                for cc in range(L):
                    col = plsc.load_gather(
                        accbuf, [lane, jnp.full((L,), cc, jnp.int32)])
                    tot = tot + col
                z = jnp.minimum(-tot, -ALPHA * tot)
                ev[pl.ds(k * L, L)] = jnp.exp(z)
                return carry2

            lax.fori_loop(0, CH // L, edge16, 0)
            off = base + i * CH
            pltpu.async_copy(mv, m_out.at[pl.ds(off, CH), :], semw[b])
            pltpu.async_copy(ev, e_out.at[pl.ds(off, CH)], semw[b])

        issue_gather(0, 0)

        def pair(g, carry):
            i0 = 2 * g
            i1 = i0 + 1

            @pl.when(i1 < n_chunks)
            def _():
                issue_gather(i1, 1)
            wait_gather(0)

            @pl.when(g > 0)
            def _():
                wait_write(0)
            compute_store(i0, 0)

            @pl.when(i0 + 2 < n_chunks)
            def _():
                issue_gather(i0 + 2, 0)

            @pl.when(i1 < n_chunks)
            def _():
                wait_gather(1)

                @pl.when(g > 0)
                def _():
                    wait_write(1)
                compute_store(i1, 1)
            return carry

        lax.fori_loop(0, (n_chunks + 1) // 2, pair, 0)
        wait_write(0)
        wait_write(1)

    return kern(src3, dst3, p, q, r, a2)


def _sc_aggregate(src4d, dst4d, e4d, rank_pad, m, zeros_n, zeros_h):
    _, ng, gsz, chw = src4d.shape
    n_nodes = 10000
    out = m.shape[1]
    n_pad = rank_pad.shape[0]
    npt = n_pad // NS
    rows_per_tile = ng * gsz
    mesh = plsc.VectorSubcoreMesh(core_axis_name="c", subcore_axis_name="s")

    @functools.partial(
        pl.kernel,
        out_type=[
            jax.ShapeDtypeStruct((2, n_nodes, out), jnp.float32),
            jax.ShapeDtypeStruct((n_nodes,), jnp.float32),
            jax.ShapeDtypeStruct((NS, ng, gsz, chw), jnp.float32),
        ],
        mesh=mesh,
        compiler_params=pltpu.CompilerParams(needs_layout_passes=False),
        scratch_types=[
            [pltpu.VMEM((gsz, CH), jnp.int32)] * 2,
            [pltpu.VMEM((gsz, CH), jnp.int32)] * 2,
            pltpu.VMEM((gsz, CH), jnp.float32),
            [pltpu.VMEM((gsz, CH), jnp.float32)] * 2,
            pltpu.VMEM((CH,), jnp.float32),
            pltpu.VMEM((n_pad,), jnp.float32),
            pltpu.VMEM((npt,), jnp.float32),
            pltpu.VMEM((npt,), jnp.float32),
            [pltpu.VMEM((CH, 128), jnp.float32)] * 2,
            pltpu.VMEM_SHARED((n_pad,), jnp.float32),
            pltpu.VMEM_SHARED((n_pad,), jnp.float32),
            pltpu.VMEM_SHARED((n_pad,), jnp.float32),
            pltpu.VMEM_SHARED((n_pad, 128), jnp.float32),
            [pltpu.SemaphoreType.DMA] * 2,
            [pltpu.SemaphoreType.DMA] * 2,
            [pltpu.SemaphoreType.DMA] * 2,
        ],
    )
    def kern(src_hbm, dst_hbm, e_hbm, rank_hbm, m_hbm, zn_hbm, zh_hbm,
             hp_out, rn_out, rel_out,
             srcg, dstg, e_b, relg, wbuf, tbl, cb1, cb2, mrow2,
             x_sp, y_sp, z_sp, h_sp, semgrp, semm, semsc):
        src_b = srcg[0]
        dst_b = dstg[0]
        rel_b = relg[0]
        c = lax.axis_index("c")
        s = lax.axis_index("s")
        row0 = s * rows_per_tile
        nd0 = s * npt
        ndsl = pl.ds(nd0, npt)

        pltpu.sync_copy(zn_hbm, x_sp.at[ndsl])
        pltpu.sync_copy(zn_hbm, y_sp.at[ndsl])
        pltpu.sync_copy(zn_hbm, z_sp.at[ndsl])
        pltpu.sync_copy(zh_hbm, h_sp.at[ndsl, :])
        plsc.subcore_barrier()

        def p1(g, carry):
            pltpu.sync_copy(e_hbm.at[s, g], e_b)
            pltpu.sync_copy(dst_hbm.at[s, g], dst_b)
            def p1j(j, carry2):
                pltpu.sync_copy(e_b.at[j], x_sp.at[dst_b.at[j]], add=True)
                return carry2
            lax.fori_loop(0, gsz, p1j, 0)
            return carry
        lax.fori_loop(0, ng, p1, 0)
        plsc.subcore_barrier()

        pltpu.sync_copy(x_sp.at[ndsl], cb1)
        def p1b(k, carry):
            sl = pl.ds(k * L, L)
            v = cb1[sl]
            v = jnp.where(v == 0.0, jnp.float32(1e-12), v)
            cb1[sl] = jnp.float32(1.0) / v
            return carry
        lax.fori_loop(0, npt // L, p1b, 0)
        pltpu.sync_copy(cb1, x_sp.at[ndsl])
        plsc.subcore_barrier()

        pltpu.sync_copy(x_sp, tbl)
        def p2(g, carry):
            pltpu.sync_copy(e_hbm.at[s, g], e_b)
            pltpu.sync_copy(dst_hbm.at[s, g], dst_b)
            pltpu.sync_copy(src_hbm.at[s, g], src_b)
            def p2j(j, carry2):
                for k in range(CH // L):
                    sl = pl.ds(k * L, L)
                    av = plsc.load_gather(tbl, [dst_b[j, sl]])
                    rel_b[j, sl] = e_b[j, sl] * av
                pltpu.sync_copy(rel_b.at[j], y_sp.at[src_b.at[j]], add=True)
                return carry2
            lax.fori_loop(0, gsz, p2j, 0)
            pltpu.sync_copy(rel_b, rel_out.at[s, g])
            return carry
        lax.fori_loop(0, ng, p2, 0)
        plsc.subcore_barrier()

        pltpu.sync_copy(y_sp.at[ndsl], cb1)
        pltpu.sync_copy(rank_hbm.at[ndsl], cb2)
        def p2b(k, carry):
            sl = pl.ds(k * L, L)
            v = cb1[sl]
            v = jnp.where(v == 0.0, jnp.float32(1e-12), v)
            cb1[sl] = cb2[sl] / v
            return carry
        lax.fori_loop(0, npt // L, p2b, 0)
        pltpu.sync_copy(cb1, y_sp.at[ndsl])
        plsc.subcore_barrier()

        pltpu.sync_copy(y_sp, tbl)
        def p3(g, carry):
            pltpu.sync_copy(rel_out.at[s, g], rel_b)
            pltpu.sync_copy(dst_hbm.at[s, g], dst_b)
            pltpu.sync_copy(src_hbm.at[s, g], src_b)
            def p3j(j, carry2):
                for k in range(CH // L):
                    sl = pl.ds(k * L, L)
                    yv = plsc.load_gather(tbl, [src_b[j, sl]])
                    e_b[j, sl] = rel_b[j, sl] * yv
                pltpu.sync_copy(e_b.at[j], z_sp.at[dst_b.at[j]], add=True)
                return carry2
            lax.fori_loop(0, gsz, p3j, 0)
            return carry
        lax.fori_loop(0, ng, p3, 0)
        plsc.subcore_barrier()

        pltpu.sync_copy(z_sp.at[ndsl], cb1)
        def p3b(k, carry):
            sl = pl.ds(k * L, L)
            cb1[sl] = jnp.float32(1.0 - DAMP) + jnp.float32(DAMP) * cb1[sl]
            return carry
        lax.fori_loop(0, npt // L, p3b, 0)
        pltpu.sync_copy(cb1, z_sp.at[ndsl])

        @pl.when(c == 0)
        def _():
            @pl.when(s < NS - 1)
            def _():
                pltpu.sync_copy(cb1, rn_out.at[ndsl])
            @pl.when(s == NS - 1)
            def _():
                rem = n_nodes - (NS - 1) * npt
                pltpu.sync_copy(cb1.at[pl.ds(0, rem)],
                                rn_out.at[pl.ds((NS - 1) * npt, rem)])
        plsc.subcore_barrier()

        pltpu.sync_copy(z_sp, tbl)

        def issue_grp(u, gb):
            pltpu.async_copy(rel_out.at[s, u], relg[gb], semgrp[gb])
            pltpu.async_copy(src_hbm.at[s, u], srcg[gb], semgrp[gb])
            pltpu.async_copy(dst_hbm.at[s, u], dstg[gb], semgrp[gb])

        def wait_grp(gb):
            pltpu.make_async_copy(rel_out.at[s, 0], relg[gb],
                                  semgrp[gb]).wait()
            pltpu.make_async_copy(src_hbm.at[s, 0], srcg[gb],
                                  semgrp[gb]).wait()
            pltpu.make_async_copy(dst_hbm.at[s, 0], dstg[gb],
                                  semgrp[gb]).wait()

        def issue_m(u, k5, mb):
            cid = u * gsz + 2 * k5 + c
            pltpu.async_copy(m_hbm.at[pl.ds((row0 + cid) * CH, CH), :],
                             mrow2[mb], semm[mb])

        def wait_m(mb):
            pltpu.make_async_copy(m_hbm.at[pl.ds(row0 * CH, CH), :],
                                  mrow2[mb], semm[mb]).wait()

        def wait_sc(mb):
            pltpu.make_async_copy(mrow2[mb], h_sp.at[dstg[0].at[0]],
                                  semsc[mb]).wait()

        def proc_group(u, gb):
            for k5 in range(5):
                mb = k5 % 2
                j = 2 * k5 + c
                wait_m(mb)
                for k in range(CH // L):
                    sl = pl.ds(k * L, L)
                    zv = plsc.load_gather(tbl, [srcg[gb][j, sl]])
                    wbuf[sl] = relg[gb][j, sl] * zv

                def rowloop(row, carry3):
                    wb = plsc.load_gather(
                        wbuf, [jnp.full((L,), row, jnp.int32)])
                    for jj in range(8):
                        fsl = pl.ds(jj * L, L)
                        mrow2[mb][row, fsl] = mrow2[mb][row, fsl] * wb
                    return carry3
                lax.fori_loop(0, CH, rowloop, 0)
                pltpu.async_copy(mrow2[mb], h_sp.at[dstg[gb].at[j]],
                                 semsc[mb], add=True)
                if k5 < 4:
                    if k5 >= 1:
                        wait_sc(1 - mb)
                    issue_m(u, k5 + 1, 1 - mb)
            wait_sc(0)
            wait_sc(1)

        issue_grp(0, 0)

        def gpair(v, carry):
            u0 = 2 * v
            u1 = u0 + 1

            @pl.when(u1 < ng)
            def _():
                issue_grp(u1, 1)
            wait_grp(0)
            issue_m(u0, 0, 0)
            proc_group(u0, 0)

            @pl.when(u0 + 2 < ng)
            def _():
                issue_grp(u0 + 2, 0)

            @pl.when(u1 < ng)
            def _():
                wait_grp(1)
                issue_m(u1, 0, 0)
                proc_group(u1, 1)
            return carry

        lax.fori_loop(0, (ng + 1) // 2, gpair, 0)
        plsc.subcore_barrier()

        @pl.when(s < NS - 1)
        def _():
            pltpu.sync_copy(h_sp.at[ndsl, :], hp_out.at[c, ndsl, :])
        @pl.when(s == NS - 1)
        def _():
            rem = n_nodes - (NS - 1) * npt
            pltpu.sync_copy(h_sp.at[pl.ds((NS - 1) * npt, rem), :],
                            hp_out.at[c, pl.ds((NS - 1) * npt, rem), :])

    return kern(src4d, dst4d, e4d, rank_pad, m, zeros_n, zeros_h)


def kernel(input, edge, edge_embed, edge_list_nhop, edge_embed_nhop,
           entity_rank, Corpus_, a, a_2):
    x = input
    n, in_f = x.shape
    edge_all = jnp.concatenate([edge, edge_list_nhop], axis=1)
    ee_all = jnp.concatenate([edge_embed, edge_embed_nhop], axis=0)
    e_total = edge_all.shape[1]
    nrela = ee_all.shape[1]
    out_f = a.shape[0]

    src = edge_all[0]
    dst = edge_all[1]

    a_srcT = a[:, :in_f].T
    a_dstT = a[:, in_f:2 * in_f].T
    a_relT = a[:, 2 * in_f:].T
    a2_vec = a_2.reshape(out_f)

    p_arr, q_arr = _tc_pq(x, a_srcT, a_dstT)
    r_arr = _tc_r(ee_all, a_relT)

    nch_a = e_total // NW // CH
    m_arr, e_arr = _sc_edge_mlp(src.reshape(NW, nch_a, CH),
                                dst.reshape(NW, nch_a, CH),
                                p_arr, q_arr, r_arr, a2_vec)

    gsz = 10
    ngrp = e_total // CH // NS // gsz
    shape4 = (NS, ngrp, gsz, CH)
    src4d = src.reshape(shape4)
    dst4d = dst.reshape(shape4)
    e4d = e_arr.reshape(shape4)
    n_pad = 10240
    rank_pad = jnp.concatenate(
        [entity_rank, jnp.zeros((n_pad - n,), jnp.float32)])
    zeros_n = jnp.zeros((n_pad // NS,), jnp.float32)
    zeros_h = jnp.zeros((n_pad // NS, out_f), jnp.float32)

    hp, rank_new, _ = _sc_aggregate(src4d, dst4d, e4d, rank_pad, m_arr,
                                    zeros_n, zeros_h)
    h_out = _tc_post(hp)
    return (h_out, rank_new)


# speedup vs baseline: 11.7175x; 1.5051x over previous
import functools

import jax
import jax.numpy as jnp
from jax import lax
from jax.experimental import pallas as pl
from jax.experimental.pallas import tpu as pltpu
from jax.experimental.pallas import tpu_sc as plsc

ALPHA = 0.2
DAMP = 0.85
NC = 2
NS = 16
NW = NC * NS
L = 16
CH = 80


def _lrelu(x):
    return jnp.maximum(x, ALPHA * x)


def _tc_pq(x, a_srcT, a_dstT):
    n, f = x.shape
    out = a_srcT.shape[1]
    bn = 2000

    def body(x_ref, as_ref, ad_ref, p_ref, q_ref):
        xb = x_ref[...]
        p_ref[...] = jnp.dot(xb, as_ref[...], preferred_element_type=jnp.float32)
        q_ref[...] = jnp.dot(xb, ad_ref[...], preferred_element_type=jnp.float32)

    return pl.pallas_call(
        body,
        grid=(n // bn,),
        in_specs=[
            pl.BlockSpec((bn, f), lambda i: (i, 0)),
            pl.BlockSpec((f, out), lambda i: (0, 0)),
            pl.BlockSpec((f, out), lambda i: (0, 0)),
        ],
        out_specs=[
            pl.BlockSpec((bn, out), lambda i: (i, 0)),
            pl.BlockSpec((bn, out), lambda i: (i, 0)),
        ],
        out_shape=[
            jax.ShapeDtypeStruct((n, out), jnp.float32),
            jax.ShapeDtypeStruct((n, out), jnp.float32),
        ],
    )(x, a_srcT, a_dstT)


def _tc_r(ee, a_relT):
    e, nr = ee.shape
    out = a_relT.shape[1]
    be = 8000

    def body(ee_ref, ar_ref, r_ref):
        r_ref[...] = jnp.dot(ee_ref[...], ar_ref[...],
                             preferred_element_type=jnp.float32)

    return pl.pallas_call(
        body,
        grid=(e // be,),
        in_specs=[
            pl.BlockSpec((be, nr), lambda i: (i, 0)),
            pl.BlockSpec((nr, out), lambda i: (0, 0)),
        ],
        out_specs=pl.BlockSpec((be, out), lambda i: (i, 0)),
        out_shape=jax.ShapeDtypeStruct((e, out), jnp.float32),
    )(ee, a_relT)


def _tc_post(hp):
    _, n, out = hp.shape
    bn = 2000

    def body(hp_ref, o_ref):
        h = hp_ref[0] + hp_ref[1]
        o_ref[...] = jnp.where(h > 0, h, jnp.exp(h) - 1.0)

    return pl.pallas_call(
        body,
        grid=(n // bn,),
        in_specs=[pl.BlockSpec((2, bn, out), lambda i: (0, i, 0))],
        out_specs=pl.BlockSpec((bn, out), lambda i: (i, 0)),
        out_shape=jax.ShapeDtypeStruct((n, out), jnp.float32),
    )(hp)


def _sc_edge_mlp(src3, dst3, p, q, r, a2):
    _, n_chunks, chw = src3.shape
    e_total = NW * n_chunks * chw
    out = p.shape[1]
    mesh = plsc.VectorSubcoreMesh(core_axis_name="c", subcore_axis_name="s")

    @functools.partial(
        pl.kernel,
        out_type=[
            jax.ShapeDtypeStruct((e_total, out), jnp.float32),
            jax.ShapeDtypeStruct((e_total,), jnp.float32),
        ],
        mesh=mesh,
        compiler_params=pltpu.CompilerParams(needs_layout_passes=False),
        scratch_types=[
            pltpu.VMEM((n_chunks, CH), jnp.int32),
            pltpu.VMEM((n_chunks, CH), jnp.int32),
            [pltpu.VMEM((CH, 128), jnp.float32)] * 2,
            [pltpu.VMEM((CH, 128), jnp.float32)] * 2,
            [pltpu.VMEM((CH, 128), jnp.float32)] * 2,
            [pltpu.VMEM((CH, 128), jnp.float32)] * 2,
            [pltpu.VMEM((CH,), jnp.float32)] * 2,
            pltpu.VMEM((128,), jnp.float32),
            pltpu.VMEM((L, L), jnp.float32),
            [pltpu.SemaphoreType.DMA] * 2,
            [pltpu.SemaphoreType.DMA] * 2,
        ],
    )
    def kern(src_hbm, dst_hbm, p_hbm, q_hbm, r_hbm, a2_hbm, m_out, e_out,
             srcb, dstb, p_v, q_v, r_v, m_v, e_v, a2_v, accbuf,
             semg, semw):
        wid = lax.axis_index("s") * NC + lax.axis_index("c")
        base = wid * n_chunks * CH
        pltpu.sync_copy(a2_hbm, a2_v)
        pltpu.sync_copy(src_hbm.at[wid], srcb)
        pltpu.sync_copy(dst_hbm.at[wid], dstb)
        lane = lax.iota(jnp.int32, L)

        def issue_gather(i, b):
            pltpu.async_copy(p_hbm.at[srcb.at[i]], p_v[b], semg[b])
            pltpu.async_copy(q_hbm.at[dstb.at[i]], q_v[b], semg[b])
            pltpu.async_copy(r_hbm.at[pl.ds(base + i * CH, CH), :],
                             r_v[b], semg[b])

        def wait_gather(b):
            pltpu.make_async_copy(p_hbm.at[srcb.at[0]], p_v[b], semg[b]).wait()
            pltpu.make_async_copy(q_hbm.at[dstb.at[0]], q_v[b], semg[b]).wait()
            pltpu.make_async_copy(r_hbm.at[pl.ds(base, CH), :],
                                  r_v[b], semg[b]).wait()

        def wait_write(b):
            pltpu.make_async_copy(m_v[b], m_out.at[pl.ds(base, CH), :],
                                  semw[b]).wait()
            pltpu.make_async_copy(e_v[b], e_out.at[pl.ds(base, CH)],
                                  semw[b]).wait()

        def compute_store(i, b):
            pv, qv, rv, mv, ev = p_v[b], q_v[b], r_v[b], m_v[b], e_v[b]

            def edge16(k, carry2):
                def one(kk):
                    e_idx = k * L + kk
                    acc = jnp.zeros((L,), jnp.float32)
                    for j in range(8):
                        sl = pl.ds(j * L, L)
                        s = pv[e_idx, sl] + qv[e_idx, sl] + rv[e_idx, sl]
                        m = _lrelu(s)
                        mv[e_idx, sl] = m
                        acc = acc + a2_v[sl] * m
                    accbuf[kk, :] = acc

                plsc.parallel_loop(0, L, unroll=4)(one)
                tot = jnp.zeros((L,), jnp.float32)
                for cc in range(L):
                    col = plsc.load_gather(
                        accbuf, [lane, jnp.full((L,), cc, jnp.int32)])
                    tot = tot + col
                z = jnp.minimum(-tot, -ALPHA * tot)
                ev[pl.ds(k * L, L)] = jnp.exp(z)
                return carry2

            lax.fori_loop(0, CH // L, edge16, 0)
            off = base + i * CH
            pltpu.async_copy(mv, m_out.at[pl.ds(off, CH), :], semw[b])
            pltpu.async_copy(ev, e_out.at[pl.ds(off, CH)], semw[b])

        issue_gather(0, 0)

        def pair(g, carry):
            i0 = 2 * g
            i1 = i0 + 1

            @pl.when(i1 < n_chunks)
            def _():
                issue_gather(i1, 1)
            wait_gather(0)

            @pl.when(g > 0)
            def _():
                wait_write(0)
            compute_store(i0, 0)

            @pl.when(i0 + 2 < n_chunks)
            def _():
                issue_gather(i0 + 2, 0)

            @pl.when(i1 < n_chunks)
            def _():
                wait_gather(1)

                @pl.when(g > 0)
                def _():
                    wait_write(1)
                compute_store(i1, 1)
            return carry

        lax.fori_loop(0, (n_chunks + 1) // 2, pair, 0)
        wait_write(0)
        wait_write(1)

    return kern(src3, dst3, p, q, r, a2)


def _sc_aggregate(src4d, dst4d, e4d, rank_pad, m, zeros_n, zeros_h):
    _, ng, gsz, chw = src4d.shape
    n_nodes = 10000
    out = m.shape[1]
    n_pad = rank_pad.shape[0]
    npt = n_pad // NS
    rows_per_tile = ng * gsz
    mesh = plsc.VectorSubcoreMesh(core_axis_name="c", subcore_axis_name="s")

    @functools.partial(
        pl.kernel,
        out_type=[
            jax.ShapeDtypeStruct((2, n_nodes, out), jnp.float32),
            jax.ShapeDtypeStruct((n_nodes,), jnp.float32),
            jax.ShapeDtypeStruct((NS, ng, gsz, chw), jnp.float32),
        ],
        mesh=mesh,
        compiler_params=pltpu.CompilerParams(needs_layout_passes=False),
        scratch_types=[
            [pltpu.VMEM((gsz, CH), jnp.int32)] * 2,
            [pltpu.VMEM((gsz, CH), jnp.int32)] * 2,
            pltpu.VMEM((gsz, CH), jnp.float32),
            [pltpu.VMEM((gsz, CH), jnp.float32)] * 2,
            pltpu.VMEM((CH,), jnp.float32),
            pltpu.VMEM((n_pad,), jnp.float32),
            pltpu.VMEM((npt,), jnp.float32),
            pltpu.VMEM((npt,), jnp.float32),
            [pltpu.VMEM((CH, 128), jnp.float32)] * 2,
            pltpu.VMEM_SHARED((n_pad,), jnp.float32),
            pltpu.VMEM_SHARED((n_pad,), jnp.float32),
            pltpu.VMEM_SHARED((n_pad,), jnp.float32),
            pltpu.VMEM_SHARED((n_pad, 128), jnp.float32),
            [pltpu.SemaphoreType.DMA] * 2,
            [pltpu.SemaphoreType.DMA] * 2,
            [pltpu.SemaphoreType.DMA] * 2,
        ],
    )
    def kern(src_hbm, dst_hbm, e_hbm, rank_hbm, m_hbm, zn_hbm, zh_hbm,
             hp_out, rn_out, rel_out,
             srcg, dstg, e_b, relg, wbuf, tbl, cb1, cb2, mrow2,
             x_sp, y_sp, z_sp, h_sp, semgrp, semm, semsc):
        src_b = srcg[0]
        dst_b = dstg[0]
        rel_b = relg[0]
        c = lax.axis_index("c")
        s = lax.axis_index("s")
        row0 = s * rows_per_tile
        nd0 = s * npt
        ndsl = pl.ds(nd0, npt)

        pltpu.sync_copy(zn_hbm, x_sp.at[ndsl])
        pltpu.sync_copy(zn_hbm, y_sp.at[ndsl])
        pltpu.sync_copy(zn_hbm, z_sp.at[ndsl])
        pltpu.sync_copy(zh_hbm, h_sp.at[ndsl, :])
        plsc.subcore_barrier()

        def p1(g, carry):
            pltpu.sync_copy(e_hbm.at[s, g], e_b)
            pltpu.sync_copy(dst_hbm.at[s, g], dst_b)
            def p1j(j, carry2):
                pltpu.sync_copy(e_b.at[j], x_sp.at[dst_b.at[j]], add=True)
                return carry2
            lax.fori_loop(0, gsz, p1j, 0)
            return carry
        lax.fori_loop(0, ng, p1, 0)
        plsc.subcore_barrier()

        pltpu.sync_copy(x_sp.at[ndsl], cb1)
        def p1b(k, carry):
            sl = pl.ds(k * L, L)
            v = cb1[sl]
            v = jnp.where(v == 0.0, jnp.float32(1e-12), v)
            cb1[sl] = jnp.float32(1.0) / v
            return carry
        lax.fori_loop(0, npt // L, p1b, 0)
        pltpu.sync_copy(cb1, x_sp.at[ndsl])
        plsc.subcore_barrier()

        pltpu.sync_copy(x_sp, tbl)
        def p2(g, carry):
            pltpu.sync_copy(e_hbm.at[s, g], e_b)
            pltpu.sync_copy(dst_hbm.at[s, g], dst_b)
            pltpu.sync_copy(src_hbm.at[s, g], src_b)
            def p2j(j, carry2):
                for k in range(CH // L):
                    sl = pl.ds(k * L, L)
                    av = plsc.load_gather(tbl, [dst_b[j, sl]])
                    rel_b[j, sl] = e_b[j, sl] * av
                pltpu.sync_copy(rel_b.at[j], y_sp.at[src_b.at[j]], add=True)
                return carry2
            lax.fori_loop(0, gsz, p2j, 0)
            pltpu.sync_copy(rel_b, rel_out.at[s, g])
            return carry
        lax.fori_loop(0, ng, p2, 0)
        plsc.subcore_barrier()

        pltpu.sync_copy(y_sp.at[ndsl], cb1)
        pltpu.sync_copy(rank_hbm.at[ndsl], cb2)
        def p2b(k, carry):
            sl = pl.ds(k * L, L)
            v = cb1[sl]
            v = jnp.where(v == 0.0, jnp.float32(1e-12), v)
            cb1[sl] = cb2[sl] / v
            return carry
        lax.fori_loop(0, npt // L, p2b, 0)
        pltpu.sync_copy(cb1, y_sp.at[ndsl])
        plsc.subcore_barrier()

        pltpu.sync_copy(y_sp, tbl)
        def p3(g, carry):
            pltpu.sync_copy(rel_out.at[s, g], rel_b)
            pltpu.sync_copy(dst_hbm.at[s, g], dst_b)
            pltpu.sync_copy(src_hbm.at[s, g], src_b)
            def p3j(j, carry2):
                for k in range(CH // L):
                    sl = pl.ds(k * L, L)
                    yv = plsc.load_gather(tbl, [src_b[j, sl]])
                    e_b[j, sl] = rel_b[j, sl] * yv
                pltpu.sync_copy(e_b.at[j], z_sp.at[dst_b.at[j]], add=True)
                return carry2
            lax.fori_loop(0, gsz, p3j, 0)
            return carry
        lax.fori_loop(0, ng, p3, 0)
        plsc.subcore_barrier()

        pltpu.sync_copy(z_sp.at[ndsl], cb1)
        def p3b(k, carry):
            sl = pl.ds(k * L, L)
            cb1[sl] = jnp.float32(1.0 - DAMP) + jnp.float32(DAMP) * cb1[sl]
            return carry
        lax.fori_loop(0, npt // L, p3b, 0)
        pltpu.sync_copy(cb1, z_sp.at[ndsl])

        @pl.when(c == 0)
        def _():
            @pl.when(s < NS - 1)
            def _():
                pltpu.sync_copy(cb1, rn_out.at[ndsl])
            @pl.when(s == NS - 1)
            def _():
                rem = n_nodes - (NS - 1) * npt
                pltpu.sync_copy(cb1.at[pl.ds(0, rem)],
                                rn_out.at[pl.ds((NS - 1) * npt, rem)])
        plsc.subcore_barrier()

        pltpu.sync_copy(z_sp, tbl)

        def issue_grp(u, gb):
            pltpu.async_copy(rel_out.at[s, u], relg[gb], semgrp[gb])
            pltpu.async_copy(src_hbm.at[s, u], srcg[gb], semgrp[gb])
            pltpu.async_copy(dst_hbm.at[s, u], dstg[gb], semgrp[gb])

        def wait_grp(gb):
            pltpu.make_async_copy(rel_out.at[s, 0], relg[gb],
                                  semgrp[gb]).wait()
            pltpu.make_async_copy(src_hbm.at[s, 0], srcg[gb],
                                  semgrp[gb]).wait()
            pltpu.make_async_copy(dst_hbm.at[s, 0], dstg[gb],
                                  semgrp[gb]).wait()

        def issue_m(u, k5, mb):
            cid = u * gsz + 2 * k5 + c
            pltpu.async_copy(m_hbm.at[pl.ds((row0 + cid) * CH, CH), :],
                             mrow2[mb], semm[mb])

        def wait_m(mb):
            pltpu.make_async_copy(m_hbm.at[pl.ds(row0 * CH, CH), :],
                                  mrow2[mb], semm[mb]).wait()

        def wait_sc(mb):
            pltpu.make_async_copy(mrow2[mb], h_sp.at[dstg[0].at[0]],
                                  semsc[mb]).wait()

        def proc_group(u, gb):
            for k5 in range(5):
                mb = k5 % 2
                j = 2 * k5 + c
                wait_m(mb)
                for k in range(CH // L):
                    sl = pl.ds(k * L, L)
                    zv = plsc.load_gather(tbl, [srcg[gb][j, sl]])
                    wbuf[sl] = relg[gb][j, sl] * zv

                def rowloop(row):
                    wb = plsc.load_gather(
                        wbuf, [jnp.full((L,), row, jnp.int32)])
                    for jj in range(8):
                        fsl = pl.ds(jj * L, L)
                        mrow2[mb][row, fsl] = mrow2[mb][row, fsl] * wb
                plsc.parallel_loop(0, CH, unroll=4)(rowloop)
                pltpu.async_copy(mrow2[mb], h_sp.at[dstg[gb].at[j]],
                                 semsc[mb], add=True)
                if k5 < 4:
                    if k5 >= 1:
                        wait_sc(1 - mb)
                    issue_m(u, k5 + 1, 1 - mb)
            wait_sc(0)
            wait_sc(1)

        issue_grp(0, 0)

        def gpair(v, carry):
            u0 = 2 * v
            u1 = u0 + 1

            @pl.when(u1 < ng)
            def _():
                issue_grp(u1, 1)
            wait_grp(0)
            issue_m(u0, 0, 0)
            proc_group(u0, 0)

            @pl.when(u0 + 2 < ng)
            def _():
                issue_grp(u0 + 2, 0)

            @pl.when(u1 < ng)
            def _():
                wait_grp(1)
                issue_m(u1, 0, 0)
                proc_group(u1, 1)
            return carry

        lax.fori_loop(0, (ng + 1) // 2, gpair, 0)
        plsc.subcore_barrier()

        @pl.when(s < NS - 1)
        def _():
            pltpu.sync_copy(h_sp.at[ndsl, :], hp_out.at[c, ndsl, :])
        @pl.when(s == NS - 1)
        def _():
            rem = n_nodes - (NS - 1) * npt
            pltpu.sync_copy(h_sp.at[pl.ds((NS - 1) * npt, rem), :],
                            hp_out.at[c, pl.ds((NS - 1) * npt, rem), :])

    return kern(src4d, dst4d, e4d, rank_pad, m, zeros_n, zeros_h)


def kernel(input, edge, edge_embed, edge_list_nhop, edge_embed_nhop,
           entity_rank, Corpus_, a, a_2):
    x = input
    n, in_f = x.shape
    edge_all = jnp.concatenate([edge, edge_list_nhop], axis=1)
    ee_all = jnp.concatenate([edge_embed, edge_embed_nhop], axis=0)
    e_total = edge_all.shape[1]
    nrela = ee_all.shape[1]
    out_f = a.shape[0]

    src = edge_all[0]
    dst = edge_all[1]

    a_srcT = a[:, :in_f].T
    a_dstT = a[:, in_f:2 * in_f].T
    a_relT = a[:, 2 * in_f:].T
    a2_vec = a_2.reshape(out_f)

    p_arr, q_arr = _tc_pq(x, a_srcT, a_dstT)
    r_arr = _tc_r(ee_all, a_relT)

    nch_a = e_total // NW // CH
    m_arr, e_arr = _sc_edge_mlp(src.reshape(NW, nch_a, CH),
                                dst.reshape(NW, nch_a, CH),
                                p_arr, q_arr, r_arr, a2_vec)

    gsz = 10
    ngrp = e_total // CH // NS // gsz
    shape4 = (NS, ngrp, gsz, CH)
    src4d = src.reshape(shape4)
    dst4d = dst.reshape(shape4)
    e4d = e_arr.reshape(shape4)
    n_pad = 10240
    rank_pad = jnp.concatenate(
        [entity_rank, jnp.zeros((n_pad - n,), jnp.float32)])
    zeros_n = jnp.zeros((n_pad // NS,), jnp.float32)
    zeros_h = jnp.zeros((n_pad // NS, out_f), jnp.float32)

    hp, rank_new, _ = _sc_aggregate(src4d, dst4d, e4d, rank_pad, m_arr,
                                    zeros_n, zeros_h)
    h_out = _tc_post(hp)
    return (h_out, rank_new)
